# baseline (speedup 1.0000x reference)
import functools

import jax
import jax.numpy as jnp
from jax import lax
from jax.experimental import pallas as pl
from jax.experimental.pallas import tpu as pltpu
from jax.experimental.pallas import tpu_sc as plsc

N = 10000
E = 320000
C = 128
NS = 16
NC = 2
CHUNK = 128
GCPT = 32
NGRP = 5
CPT = GCPT * NGRP
EP = NS * CPT * CHUNK
NODES_PER_TILE = 640
NACC = NS * NODES_PER_TILE
DUMMY = 10016
LAST_TILE_ROWS = N - 15 * NODES_PER_TILE

_mesh = plsc.VectorSubcoreMesh(core_axis_name="c", subcore_axis_name="s")


def _zero_acc(zeros_h, acc, sid, width):
    for k in range(NODES_PER_TILE // CHUNK):
        pltpu.sync_copy(
            zeros_h, acc.at[pl.ds(sid * NODES_PER_TILE + k * CHUNK, CHUNK)])


def _dump_acc(acc, out, sid):
    @pl.when(sid < NS - 1)
    def _():
        pltpu.sync_copy(acc.at[pl.ds(sid * NODES_PER_TILE, NODES_PER_TILE)],
                        out.at[pl.ds(sid * NODES_PER_TILE, NODES_PER_TILE)])

    @pl.when(sid == NS - 1)
    def _():
        pltpu.sync_copy(acc.at[pl.ds(sid * NODES_PER_TILE, LAST_TILE_ROWS)],
                        out.at[pl.ds(sid * NODES_PER_TILE, LAST_TILE_ROWS)])


@functools.partial(
    pl.kernel,
    mesh=_mesh,
    out_type=[jax.ShapeDtypeStruct((N, 16), jnp.float32),
              jax.ShapeDtypeStruct((N, 16), jnp.float32)],
    scratch_types=[pltpu.VMEM((CPT, CHUNK), jnp.int32),
                   pltpu.VMEM((CHUNK, 16), jnp.float32),
                   pltpu.VMEM_SHARED((NACC, 16), jnp.float32)],
)
def _deg_kernel(rows3, cols3, zeros16_h, ones16_h, dego_out, degi_out,
                idx_v, ones_v, acc):
    cid = lax.axis_index("c")
    sid = lax.axis_index("s")
    _zero_acc(zeros16_h, acc, sid, 16)
    pltpu.sync_copy(ones16_h, ones_v)

    @pl.when(cid == 0)
    def _():
        pltpu.sync_copy(rows3.at[sid], idx_v)

    @pl.when(cid == 1)
    def _():
        pltpu.sync_copy(cols3.at[sid], idx_v)

    plsc.subcore_barrier()

    def body(j, carry):
        pltpu.sync_copy(ones_v, acc.at[idx_v.at[j]], add=True)
        return carry

    lax.fori_loop(0, CPT, body, 0)
    plsc.subcore_barrier()

    @pl.when(cid == 0)
    def _():
        _dump_acc(acc, dego_out, sid)

    @pl.when(cid == 1)
    def _():
        _dump_acc(acc, degi_out, sid)


def _make_prop(n_per_core):
    n_srcs = 2 * n_per_core
    out_type = [jax.ShapeDtypeStruct((N, C), jnp.float32)] * n_srcs
    scratch = [pltpu.VMEM((GCPT, CHUNK), jnp.int32),
               pltpu.VMEM((GCPT, CHUNK), jnp.int32),
               pltpu.VMEM((CHUNK, C), jnp.float32),
               pltpu.VMEM((CHUNK, C), jnp.float32),
               pltpu.SemaphoreType.DMA,
               pltpu.SemaphoreType.DMA,
               pltpu.SemaphoreType.DMA,
               pltpu.SemaphoreType.DMA,
               pltpu.VMEM_SHARED((NACC, C), jnp.float32)]

    @functools.partial(pl.kernel, mesh=_mesh, out_type=out_type,
                       scratch_types=scratch)
    def prop(*refs):
        rows3, cols3, zeros_h = refs[0], refs[1], refs[2]
        srcs = refs[3:3 + n_srcs]
        outs = refs[3 + n_srcs:3 + 2 * n_srcs]
        (row_v, col_v, buf0, buf1, sg0, sg1, ss0, ss1,
         acc) = refs[3 + 2 * n_srcs:]
        cid = lax.axis_index("c")
        sid = lax.axis_index("s")
        npair = GCPT // 2

        _zero_acc(zeros_h, acc, sid, C)
        plsc.subcore_barrier()

        def one_source(src, out):
            def group(g, carry):
                pltpu.sync_copy(rows3.at[sid, pl.ds(g * GCPT, GCPT)], row_v)
                pltpu.sync_copy(cols3.at[sid, pl.ds(g * GCPT, GCPT)], col_v)
                pltpu.async_copy(src.at[row_v.at[0]], buf0, sg0)
                pltpu.async_copy(src.at[row_v.at[1]], buf1, sg1)

                def pair(p, carry2):
                    c0 = 2 * p
                    c1 = c0 + 1
                    pltpu.make_async_copy(src.at[row_v.at[c0]], buf0,
                                          sg0).wait()
                    pltpu.async_copy(buf0, acc.at[col_v.at[c0]], ss0,
                                     add=True)
                    pltpu.make_async_copy(src.at[row_v.at[c1]], buf1,
                                          sg1).wait()
                    pltpu.async_copy(buf1, acc.at[col_v.at[c1]], ss1,
                                     add=True)
                    pltpu.make_async_copy(buf0, acc.at[col_v.at[c0]],
                                          ss0).wait()

                    @pl.when(p < npair - 1)
                    def _():
                        pltpu.async_copy(src.at[row_v.at[c0 + 2]], buf0, sg0)

                    pltpu.make_async_copy(buf1, acc.at[col_v.at[c1]],
                                          ss1).wait()

                    @pl.when(p < npair - 1)
                    def _():
                        pltpu.async_copy(src.at[row_v.at[c1 + 2]], buf1, sg1)

                    return carry2

                lax.fori_loop(0, npair, pair, carry)
                return carry

            lax.fori_loop(0, NGRP, group, 0)
            plsc.subcore_barrier()
            _dump_acc(acc, out, sid)
            _zero_acc(zeros_h, acc, sid, C)
            plsc.subcore_barrier()

        for s in range(n_per_core):
            @pl.when(cid == 0)
            def _(s=s):
                one_source(srcs[s], outs[s])

            @pl.when(cid == 1)
            def _(s=s):
                one_source(srcs[n_per_core + s], outs[n_per_core + s])

    return prop


_prop2 = _make_prop(1)
_prop4 = _make_prop(2)


BM = 400
GRID = N // BM


def _row_spec(width=C):
    return pl.BlockSpec((BM, width), lambda i: (i, 0))


def _full_spec(shape):
    nd = len(shape)
    return pl.BlockSpec(shape, lambda i: (0,) * nd)


def _prescale(n, xs, dego16, degi16):
    def body(*refs):
        x_refs = refs[:n]
        do_ref, di_ref = refs[n], refs[n + 1]
        o_refs = refs[n + 2:]
        rdo = 1.0 / do_ref[:, 0:1]
        rdi = 1.0 / di_ref[:, 0:1]
        for k in range(n):
            o_refs[k][...] = x_refs[k][...] * (rdo if k % 2 == 0 else rdi)

    return pl.pallas_call(
        body,
        grid=(GRID,),
        in_specs=[_row_spec() for _ in range(n)] + [_row_spec(16), _row_spec(16)],
        out_specs=[_row_spec() for _ in range(n)],
        out_shape=[jax.ShapeDtypeStruct((N, C), jnp.float32)] * n,
    )(*xs, dego16, degi16)


def _gates(X, A1o, A1i, A2o, A2i, H, B1o, B1i, B2o, B2i, dego16, degi16,
           Wzr, bzr, Whx):
    def body(x, a1o, a1i, a2o, a2i, h, b1o, b1i, b2o, b2i, do_, di_,
             wzr, bzr_, whx, z_o, g_o, go_o, gi_o, ph1_o):
        srcs = (x, a1o, a1i, a2o, a2i, h, b1o, b1i, b2o, b2i)
        acc = jnp.broadcast_to(bzr_[...], (BM, 2 * C)).astype(jnp.float32)
        for k in range(10):
            acc = acc + jnp.dot(srcs[k][...], wzr[k],
                                preferred_element_type=jnp.float32)
        zr = 1.0 / (1.0 + jnp.exp(-acc))
        z = zr[:, :C]
        r = zr[:, C:]
        g = h[...] * r
        z_o[...] = z
        g_o[...] = g
        go_o[...] = g * (1.0 / do_[:, 0:1])
        gi_o[...] = g * (1.0 / di_[:, 0:1])
        ph1 = jnp.dot(srcs[0][...], whx[0], preferred_element_type=jnp.float32)
        for k in range(1, 5):
            ph1 = ph1 + jnp.dot(srcs[k][...], whx[k],
                                preferred_element_type=jnp.float32)
        ph1_o[...] = ph1

    return pl.pallas_call(
        body,
        grid=(GRID,),
        in_specs=[_row_spec() for _ in range(10)]
        + [_row_spec(16), _row_spec(16),
           _full_spec((10, C, 2 * C)), _full_spec((1, 2 * C)),
           _full_spec((5, C, C))],
        out_specs=[_row_spec() for _ in range(5)],
        out_shape=[jax.ShapeDtypeStruct((N, C), jnp.float32)] * 5,
    )(X, A1o, A1i, A2o, A2i, H, B1o, B1i, B2o, B2i, dego16, degi16,
      Wzr, bzr, Whx)


def _final(preH1, G, C1o, C1i, C2o, C2i, H, Z, Whg, bh):
    def body(ph1, g, c1o, c1i, c2o, c2i, h, z, whg, bh_,
             hn_o, psum_o, pcnt_o):
        srcs = (g, c1o, c1i, c2o, c2i)
        acc = ph1[...] + jnp.broadcast_to(bh_[...], (BM, C))
        for k in range(5):
            acc = acc + jnp.dot(srcs[k][...], whg[k],
                                preferred_element_type=jnp.float32)
        ht = jnp.tanh(acc)
        hn = z[...] * h[...] + (1.0 - z[...]) * ht
        hn_o[...] = hn
        nanmask = jnp.isnan(hn)
        psum_o[...] = jnp.sum(
            jnp.where(nanmask, 0.0, hn).reshape(BM // 8, 8, C), axis=0,
            keepdims=True)
        pcnt_o[...] = jnp.sum(
            jnp.where(nanmask, 0.0, 1.0).reshape(BM // 8, 8, C), axis=0,
            keepdims=True)

    return pl.pallas_call(
        body,
        grid=(GRID,),
        in_specs=[_row_spec() for _ in range(8)]
        + [_full_spec((5, C, C)), _full_spec((1, C))],
        out_specs=[_row_spec(), pl.BlockSpec((1, 8, C), lambda i: (i, 0, 0)),
                   pl.BlockSpec((1, 8, C), lambda i: (i, 0, 0))],
        out_shape=[jax.ShapeDtypeStruct((N, C), jnp.float32),
                   jax.ShapeDtypeStruct((GRID, 8, C), jnp.float32),
                   jax.ShapeDtypeStruct((GRID, 8, C), jnp.float32)],
    )(preH1, G, C1o, C1i, C2o, C2i, H, Z, Whg, bh)


def _fix(hn_raw, psum, pcnt):
    def body(hn, ps, pc, out):
        mean = jnp.sum(ps[...]) / jnp.sum(pc[...])
        v = hn[...]
        out[...] = jnp.where(jnp.isnan(v), mean, v)

    return pl.pallas_call(
        body,
        out_shape=jax.ShapeDtypeStruct((N, C), jnp.float32),
    )(hn_raw, psum, pcnt)


def _combine_weights(W):
    Wx, Wh_ = W[:, :, :C, :], W[:, :, C:, :]
    xs = jnp.stack([Wx[0, 0] + Wx[1, 0] - Wx[0, 2] - Wx[1, 2],
                    Wx[0, 1], Wx[1, 1], 2.0 * Wx[0, 2], 2.0 * Wx[1, 2]])
    hs = jnp.stack([Wh_[0, 0] + Wh_[1, 0] - Wh_[0, 2] - Wh_[1, 2],
                    Wh_[0, 1], Wh_[1, 1], 2.0 * Wh_[0, 2], 2.0 * Wh_[1, 2]])
    return jnp.concatenate([xs, hs], axis=0)


def kernel(X, edge_index, edge_weight, H, Wz, bz, Wr, br, Wh, bh):
    row = edge_index[0]
    col = edge_index[1]
    pad = EP - E
    row_p = jnp.concatenate([row, jnp.zeros((pad,), jnp.int32)])
    col_p = jnp.concatenate([col, jnp.full((pad,), DUMMY, jnp.int32)])
    rows3 = row_p.reshape(NS, CPT, CHUNK)
    cols3 = col_p.reshape(NS, CPT, CHUNK)
    zeros_h = jnp.zeros((CHUNK, C), jnp.float32)
    zeros16_h = jnp.zeros((CHUNK, 16), jnp.float32)
    ones16_h = jnp.ones((CHUNK, 16), jnp.float32)

    Wz10 = _combine_weights(Wz)
    Wr10 = _combine_weights(Wr)
    Wzr = jnp.concatenate([Wz10, Wr10], axis=2)
    bzr = jnp.concatenate([bz, br]).reshape(1, 2 * C)
    Wh10 = _combine_weights(Wh)
    Whx = Wh10[:5]
    Whg = Wh10[5:]
    bh2 = bh.reshape(1, C)

    dego16, degi16 = _deg_kernel(rows3, cols3, zeros16_h, ones16_h)

    Xo, Xi, Ho, Hi = _prescale(4, (X, X, H, H), dego16, degi16)
    A1o, A1i, B1o, B1i = _prop4(rows3, cols3, zeros_h, Xo, Xi, Ho, Hi)
    A1oo, A1ii, B1oo, B1ii = _prescale(4, (A1o, A1i, B1o, B1i),
                                       dego16, degi16)
    A2o, A2i, B2o, B2i = _prop4(rows3, cols3, zeros_h, A1oo, A1ii, B1oo, B1ii)

    Z, G, Go, Gi, preH1 = _gates(X, A1o, A1i, A2o, A2i, H, B1o, B1i,
                                 B2o, B2i, dego16, degi16, Wzr, bzr, Whx)

    C1o, C1i = _prop2(rows3, cols3, zeros_h, Go, Gi)
    C1oo, C1ii = _prescale(2, (C1o, C1i), dego16, degi16)
    C2o, C2i = _prop2(rows3, cols3, zeros_h, C1oo, C1ii)

    hn_raw, psum, pcnt = _final(preH1, G, C1o, C1i, C2o, C2i, H, Z, Whg, bh2)
    return _fix(hn_raw, psum, pcnt)


# speedup vs baseline: 1.0821x; 1.0821x over previous
import functools

import jax
import jax.numpy as jnp
from jax import lax
from jax.experimental import pallas as pl
from jax.experimental.pallas import tpu as pltpu
from jax.experimental.pallas import tpu_sc as plsc

N = 10000
E = 320000
C = 128
NS = 16
NC = 2
CHUNK = 128
GCPT = 32
NGRP = 5
CPT = GCPT * NGRP
EP = NS * CPT * CHUNK
NODES_PER_TILE = 640
NACC = NS * NODES_PER_TILE
DUMMY = 10016
LAST_TILE_ROWS = N - 15 * NODES_PER_TILE

_mesh = plsc.VectorSubcoreMesh(core_axis_name="c", subcore_axis_name="s")


def _zero_acc(zeros_h, acc, sid, width):
    for k in range(NODES_PER_TILE // CHUNK):
        pltpu.sync_copy(
            zeros_h, acc.at[pl.ds(sid * NODES_PER_TILE + k * CHUNK, CHUNK)])


def _dump_acc(acc, out, sid):
    @pl.when(sid < NS - 1)
    def _():
        pltpu.sync_copy(acc.at[pl.ds(sid * NODES_PER_TILE, NODES_PER_TILE)],
                        out.at[pl.ds(sid * NODES_PER_TILE, NODES_PER_TILE)])

    @pl.when(sid == NS - 1)
    def _():
        pltpu.sync_copy(acc.at[pl.ds(sid * NODES_PER_TILE, LAST_TILE_ROWS)],
                        out.at[pl.ds(sid * NODES_PER_TILE, LAST_TILE_ROWS)])


@functools.partial(
    pl.kernel,
    mesh=_mesh,
    out_type=[jax.ShapeDtypeStruct((N, 16), jnp.float32),
              jax.ShapeDtypeStruct((N, 16), jnp.float32)],
    scratch_types=[pltpu.VMEM((CPT, CHUNK), jnp.int32),
                   pltpu.VMEM((CHUNK, 16), jnp.float32),
                   pltpu.VMEM_SHARED((NACC, 16), jnp.float32)],
)
def _deg_kernel(rows3, cols3, zeros16_h, ones16_h, dego_out, degi_out,
                idx_v, ones_v, acc):
    cid = lax.axis_index("c")
    sid = lax.axis_index("s")
    _zero_acc(zeros16_h, acc, sid, 16)
    pltpu.sync_copy(ones16_h, ones_v)

    @pl.when(cid == 0)
    def _():
        pltpu.sync_copy(rows3.at[sid], idx_v)

    @pl.when(cid == 1)
    def _():
        pltpu.sync_copy(cols3.at[sid], idx_v)

    plsc.subcore_barrier()

    def body(j, carry):
        pltpu.sync_copy(ones_v, acc.at[idx_v.at[j]], add=True)
        return carry

    lax.fori_loop(0, CPT, body, 0)
    plsc.subcore_barrier()

    @pl.when(cid == 0)
    def _():
        _dump_acc(acc, dego_out, sid)

    @pl.when(cid == 1)
    def _():
        _dump_acc(acc, degi_out, sid)


def _make_prop(n_per_core):
    n_srcs = 2 * n_per_core
    out_type = [jax.ShapeDtypeStruct((N, C), jnp.float32)] * n_srcs
    scratch = [pltpu.VMEM((GCPT, CHUNK), jnp.int32),
               pltpu.VMEM((GCPT, CHUNK), jnp.int32),
               pltpu.VMEM((CHUNK, C), jnp.float32),
               pltpu.VMEM((CHUNK, C), jnp.float32),
               pltpu.SemaphoreType.DMA,
               pltpu.SemaphoreType.DMA,
               pltpu.SemaphoreType.DMA,
               pltpu.SemaphoreType.DMA,
               pltpu.VMEM_SHARED((NACC, C), jnp.float32)]

    @functools.partial(pl.kernel, mesh=_mesh, out_type=out_type,
                       scratch_types=scratch)
    def prop(*refs):
        rows3, cols3, zeros_h = refs[0], refs[1], refs[2]
        srcs = refs[3:3 + n_srcs]
        outs = refs[3 + n_srcs:3 + 2 * n_srcs]
        (row_v, col_v, buf0, buf1, sg0, sg1, ss0, ss1,
         acc) = refs[3 + 2 * n_srcs:]
        cid = lax.axis_index("c")
        sid = lax.axis_index("s")
        npair = GCPT // 2

        _zero_acc(zeros_h, acc, sid, C)
        plsc.subcore_barrier()

        def one_source(src, out):
            def group(g, carry):
                pltpu.sync_copy(rows3.at[sid, pl.ds(g * GCPT, GCPT)], row_v)
                pltpu.sync_copy(cols3.at[sid, pl.ds(g * GCPT, GCPT)], col_v)
                pltpu.async_copy(src.at[row_v.at[0]], buf0, sg0)

                def pair(p, carry2):
                    c0 = 2 * p
                    c1 = c0 + 1
                    pltpu.async_copy(src.at[row_v.at[c1]], buf1, sg1)
                    pltpu.make_async_copy(src.at[row_v.at[c0]], buf0,
                                          sg0).wait()
                    pltpu.async_copy(buf0, acc.at[col_v.at[c0]], ss0,
                                     add=True)
                    pltpu.make_async_copy(src.at[row_v.at[c1]], buf1,
                                          sg1).wait()
                    pltpu.make_async_copy(buf0, acc.at[col_v.at[c0]],
                                          ss0).wait()

                    @pl.when(p < npair - 1)
                    def _():
                        pltpu.async_copy(src.at[row_v.at[c0 + 2]], buf0, sg0)

                    pltpu.async_copy(buf1, acc.at[col_v.at[c1]], ss1,
                                     add=True)
                    pltpu.make_async_copy(buf1, acc.at[col_v.at[c1]],
                                          ss1).wait()
                    return carry2

                lax.fori_loop(0, npair, pair, carry)
                return carry

            lax.fori_loop(0, NGRP, group, 0)
            plsc.subcore_barrier()
            _dump_acc(acc, out, sid)
            _zero_acc(zeros_h, acc, sid, C)
            plsc.subcore_barrier()

        for s in range(n_per_core):
            @pl.when(cid == 0)
            def _(s=s):
                one_source(srcs[s], outs[s])

            @pl.when(cid == 1)
            def _(s=s):
                one_source(srcs[n_per_core + s], outs[n_per_core + s])

    return prop


_prop2 = _make_prop(1)
_prop4 = _make_prop(2)


BM = 400
GRID = N // BM


def _row_spec(width=C):
    return pl.BlockSpec((BM, width), lambda i: (i, 0))


def _full_spec(shape):
    nd = len(shape)
    return pl.BlockSpec(shape, lambda i: (0,) * nd)


def _prescale(n, xs, dego16, degi16):
    def body(*refs):
        x_refs = refs[:n]
        do_ref, di_ref = refs[n], refs[n + 1]
        o_refs = refs[n + 2:]
        rdo = 1.0 / do_ref[:, 0:1]
        rdi = 1.0 / di_ref[:, 0:1]
        for k in range(n):
            o_refs[k][...] = x_refs[k][...] * (rdo if k % 2 == 0 else rdi)

    return pl.pallas_call(
        body,
        grid=(GRID,),
        in_specs=[_row_spec() for _ in range(n)] + [_row_spec(16), _row_spec(16)],
        out_specs=[_row_spec() for _ in range(n)],
        out_shape=[jax.ShapeDtypeStruct((N, C), jnp.float32)] * n,
    )(*xs, dego16, degi16)


def _gates(X, A1o, A1i, A2o, A2i, H, B1o, B1i, B2o, B2i, dego16, degi16,
           Wzr, bzr, Whx):
    def body(x, a1o, a1i, a2o, a2i, h, b1o, b1i, b2o, b2i, do_, di_,
             wzr, bzr_, whx, z_o, g_o, go_o, gi_o, ph1_o):
        srcs = (x, a1o, a1i, a2o, a2i, h, b1o, b1i, b2o, b2i)
        acc = jnp.broadcast_to(bzr_[...], (BM, 2 * C)).astype(jnp.float32)
        for k in range(10):
            acc = acc + jnp.dot(srcs[k][...], wzr[k],
                                preferred_element_type=jnp.float32)
        zr = 1.0 / (1.0 + jnp.exp(-acc))
        z = zr[:, :C]
        r = zr[:, C:]
        g = h[...] * r
        z_o[...] = z
        g_o[...] = g
        go_o[...] = g * (1.0 / do_[:, 0:1])
        gi_o[...] = g * (1.0 / di_[:, 0:1])
        ph1 = jnp.dot(srcs[0][...], whx[0], preferred_element_type=jnp.float32)
        for k in range(1, 5):
            ph1 = ph1 + jnp.dot(srcs[k][...], whx[k],
                                preferred_element_type=jnp.float32)
        ph1_o[...] = ph1

    return pl.pallas_call(
        body,
        grid=(GRID,),
        in_specs=[_row_spec() for _ in range(10)]
        + [_row_spec(16), _row_spec(16),
           _full_spec((10, C, 2 * C)), _full_spec((1, 2 * C)),
           _full_spec((5, C, C))],
        out_specs=[_row_spec() for _ in range(5)],
        out_shape=[jax.ShapeDtypeStruct((N, C), jnp.float32)] * 5,
    )(X, A1o, A1i, A2o, A2i, H, B1o, B1i, B2o, B2i, dego16, degi16,
      Wzr, bzr, Whx)


def _final(preH1, G, C1o, C1i, C2o, C2i, H, Z, Whg, bh):
    def body(ph1, g, c1o, c1i, c2o, c2i, h, z, whg, bh_,
             hn_o, psum_o, pcnt_o):
        srcs = (g, c1o, c1i, c2o, c2i)
        acc = ph1[...] + jnp.broadcast_to(bh_[...], (BM, C))
        for k in range(5):
            acc = acc + jnp.dot(srcs[k][...], whg[k],
                                preferred_element_type=jnp.float32)
        ht = jnp.tanh(acc)
        hn = z[...] * h[...] + (1.0 - z[...]) * ht
        hn_o[...] = hn
        nanmask = jnp.isnan(hn)
        psum_o[...] = jnp.sum(
            jnp.where(nanmask, 0.0, hn).reshape(BM // 8, 8, C), axis=0,
            keepdims=True)
        pcnt_o[...] = jnp.sum(
            jnp.where(nanmask, 0.0, 1.0).reshape(BM // 8, 8, C), axis=0,
            keepdims=True)

    return pl.pallas_call(
        body,
        grid=(GRID,),
        in_specs=[_row_spec() for _ in range(8)]
        + [_full_spec((5, C, C)), _full_spec((1, C))],
        out_specs=[_row_spec(), pl.BlockSpec((1, 8, C), lambda i: (i, 0, 0)),
                   pl.BlockSpec((1, 8, C), lambda i: (i, 0, 0))],
        out_shape=[jax.ShapeDtypeStruct((N, C), jnp.float32),
                   jax.ShapeDtypeStruct((GRID, 8, C), jnp.float32),
                   jax.ShapeDtypeStruct((GRID, 8, C), jnp.float32)],
    )(preH1, G, C1o, C1i, C2o, C2i, H, Z, Whg, bh)


def _fix(hn_raw, psum, pcnt):
    def body(hn, ps, pc, out):
        mean = jnp.sum(ps[...]) / jnp.sum(pc[...])
        v = hn[...]
        out[...] = jnp.where(jnp.isnan(v), mean, v)

    return pl.pallas_call(
        body,
        out_shape=jax.ShapeDtypeStruct((N, C), jnp.float32),
    )(hn_raw, psum, pcnt)


def _combine_weights(W):
    Wx, Wh_ = W[:, :, :C, :], W[:, :, C:, :]
    xs = jnp.stack([Wx[0, 0] + Wx[1, 0] - Wx[0, 2] - Wx[1, 2],
                    Wx[0, 1], Wx[1, 1], 2.0 * Wx[0, 2], 2.0 * Wx[1, 2]])
    hs = jnp.stack([Wh_[0, 0] + Wh_[1, 0] - Wh_[0, 2] - Wh_[1, 2],
                    Wh_[0, 1], Wh_[1, 1], 2.0 * Wh_[0, 2], 2.0 * Wh_[1, 2]])
    return jnp.concatenate([xs, hs], axis=0)


def kernel(X, edge_index, edge_weight, H, Wz, bz, Wr, br, Wh, bh):
    row = edge_index[0]
    col = edge_index[1]
    pad = EP - E
    row_p = jnp.concatenate([row, jnp.zeros((pad,), jnp.int32)])
    col_p = jnp.concatenate([col, jnp.full((pad,), DUMMY, jnp.int32)])
    rows3 = row_p.reshape(NS, CPT, CHUNK)
    cols3 = col_p.reshape(NS, CPT, CHUNK)
    zeros_h = jnp.zeros((CHUNK, C), jnp.float32)
    zeros16_h = jnp.zeros((CHUNK, 16), jnp.float32)
    ones16_h = jnp.ones((CHUNK, 16), jnp.float32)

    Wz10 = _combine_weights(Wz)
    Wr10 = _combine_weights(Wr)
    Wzr = jnp.concatenate([Wz10, Wr10], axis=2)
    bzr = jnp.concatenate([bz, br]).reshape(1, 2 * C)
    Wh10 = _combine_weights(Wh)
    Whx = Wh10[:5]
    Whg = Wh10[5:]
    bh2 = bh.reshape(1, C)

    dego16, degi16 = _deg_kernel(rows3, cols3, zeros16_h, ones16_h)

    Xo, Xi, Ho, Hi = _prescale(4, (X, X, H, H), dego16, degi16)
    A1o, A1i, B1o, B1i = _prop4(rows3, cols3, zeros_h, Xo, Xi, Ho, Hi)
    A1oo, A1ii, B1oo, B1ii = _prescale(4, (A1o, A1i, B1o, B1i),
                                       dego16, degi16)
    A2o, A2i, B2o, B2i = _prop4(rows3, cols3, zeros_h, A1oo, A1ii, B1oo, B1ii)

    Z, G, Go, Gi, preH1 = _gates(X, A1o, A1i, A2o, A2i, H, B1o, B1i,
                                 B2o, B2i, dego16, degi16, Wzr, bzr, Whx)

    C1o, C1i = _prop2(rows3, cols3, zeros_h, Go, Gi)
    C1oo, C1ii = _prescale(2, (C1o, C1i), dego16, degi16)
    C2o, C2i = _prop2(rows3, cols3, zeros_h, C1oo, C1ii)

    hn_raw, psum, pcnt = _final(preH1, G, C1o, C1i, C2o, C2i, H, Z, Whg, bh2)
    return _fix(hn_raw, psum, pcnt)
